# baseline (speedup 1.0000x reference)
import functools

import jax
import jax.numpy as jnp
from jax import lax
from jax.experimental import pallas as pl
from jax.experimental.pallas import tpu as pltpu
from jax.experimental.pallas import tpu_sc as plsc

NC = 2
NS = 16
NW = NC * NS
CHUNK = 128
GPS = 2
SLOT = CHUNK * GPS
D = 128
NB = 3
LOOKAHEAD = 2


@functools.cache
def _emb_kernel(n_idx: int):
  b_per_w = n_idx // NW
  n_macro = b_per_w // SLOT
  mesh = plsc.VectorSubcoreMesh(
      core_axis_name="c", subcore_axis_name="s", num_cores=NC, num_subcores=NS
  )

  @functools.partial(
      pl.kernel,
      out_type=jax.ShapeDtypeStruct((n_idx, D), jnp.float32),
      mesh=mesh,
      scratch_types=[
          pltpu.VMEM((b_per_w,), jnp.int32),
          pltpu.VMEM((NB, SLOT, D), jnp.float32),
          [pltpu.SemaphoreType.DMA for _ in range(NB)],
          [pltpu.SemaphoreType.DMA for _ in range(NB)],
      ],
  )
  def k(words_hbm, table_hbm, out_hbm, idx_v, rows_v, gsems, osems):
    wid = lax.axis_index("s") * NC + lax.axis_index("c")
    base = wid * b_per_w
    pltpu.sync_copy(words_hbm.at[pl.ds(base, b_per_w)], idx_v)

    def fire_gathers(m, slot):
      for g in range(GPS):
        pltpu.async_copy(
            table_hbm.at[idx_v.at[pl.ds(m * SLOT + g * CHUNK, CHUNK)]],
            rows_v.at[slot].at[pl.ds(g * CHUNK, CHUNK)],
            gsems[slot],
        )

    def wait_gathers(slot):
      pltpu.make_async_copy(
          table_hbm.at[pl.ds(0, SLOT)], rows_v.at[slot], gsems[slot]
      ).wait()

    def fire_write(m, slot):
      pltpu.async_copy(
          rows_v.at[slot], out_hbm.at[pl.ds(base + m * SLOT, SLOT)],
          osems[slot],
      )

    def wait_write(slot):
      pltpu.make_async_copy(
          table_hbm.at[pl.ds(0, SLOT)], rows_v.at[slot], osems[slot]
      ).wait()

    for m in range(LOOKAHEAD):
      fire_gathers(m, m % NB)

    n_main = (n_macro // NB) * NB

    def outer(o, _):
      for b in range(NB):
        m = o * NB + b
        wait_gathers(b)
        fire_write(m, b)
        nb = (b + LOOKAHEAD) % NB

        @pl.when(m + LOOKAHEAD < n_macro)
        def _():
          @pl.when(m + LOOKAHEAD >= NB)
          def _():
            wait_write(nb)

          fire_gathers(m + LOOKAHEAD, nb)

      return 0

    lax.fori_loop(0, n_main // NB, outer, 0)

    for m in range(n_main, n_macro):
      b = m % NB
      wait_gathers(b)
      fire_write(m, b)
    for m in range(n_macro - NB, n_macro):
      wait_write(m % NB)

  return k


def kernel(words, table):
  b, h = words.shape
  idx = words.T.reshape(-1).astype(jnp.int32)
  out = _emb_kernel(idx.size)(idx, table.astype(jnp.float32))
  return out.reshape(h, b, D).transpose(1, 0, 2)


# speedup vs baseline: 1.0126x; 1.0126x over previous
import functools

import jax
import jax.numpy as jnp
from jax import lax
from jax.experimental import pallas as pl
from jax.experimental.pallas import tpu as pltpu
from jax.experimental.pallas import tpu_sc as plsc

NC = 2
NS = 16
NW = NC * NS
CHUNK = 128
D = 128
NB = 5
LOOKAHEAD = 3


@functools.cache
def _emb_kernel(n_idx: int):
  b_per_w = n_idx // NW
  n_chunks = b_per_w // CHUNK
  mesh = plsc.VectorSubcoreMesh(
      core_axis_name="c", subcore_axis_name="s", num_cores=NC, num_subcores=NS
  )

  @functools.partial(
      pl.kernel,
      out_type=jax.ShapeDtypeStruct((n_idx, D), jnp.float32),
      mesh=mesh,
      compiler_params=pltpu.CompilerParams(
          disable_bounds_checks=True, disable_semaphore_checks=True
      ),
      scratch_types=[
          pltpu.VMEM((b_per_w,), jnp.int32),
          pltpu.VMEM((NB * CHUNK, D), jnp.float32),
          pltpu.SemaphoreType.DMA((NB,)),
          pltpu.SemaphoreType.DMA((NB,)),
      ],
  )
  def k(words_hbm, table_hbm, out_hbm, idx_v, rows_v, gsem, osem):
    wid = lax.axis_index("s") * NC + lax.axis_index("c")
    base = wid * b_per_w
    pltpu.sync_copy(words_hbm.at[pl.ds(base, b_per_w)], idx_v)

    def slot_ref(s):
      return rows_v.at[pl.ds(s * CHUNK, CHUNK)]

    def fire_gather(c, s):
      pltpu.async_copy(
          table_hbm.at[idx_v.at[pl.ds(c * CHUNK, CHUNK)]],
          slot_ref(s),
          gsem.at[s],
      )

    def wait_write(s):
      pltpu.make_async_copy(
          table_hbm.at[pl.ds(0, CHUNK)], slot_ref(s), osem.at[s]
      ).wait()

    for c in range(LOOKAHEAD):
      fire_gather(c, c % NB)

    def body(j, _):
      s = lax.rem(j, NB)
      pltpu.make_async_copy(
          table_hbm.at[pl.ds(0, CHUNK)], slot_ref(s), gsem.at[s]
      ).wait()
      pltpu.async_copy(
          slot_ref(s), out_hbm.at[pl.ds(base + j * CHUNK, CHUNK)], osem.at[s]
      )
      nxt = j + LOOKAHEAD
      ns = lax.rem(nxt, NB)

      @pl.when(nxt < n_chunks)
      def _():
        @pl.when(nxt >= NB)
        def _():
          wait_write(ns)

        fire_gather(nxt, ns)

      return 0

    lax.fori_loop(0, n_chunks, body, 0)

    for b in range(NB):
      wait_write(b)

  return k


def kernel(words, table):
  b, h = words.shape
  idx = words.T.reshape(-1).astype(jnp.int32)
  out = _emb_kernel(idx.size)(idx, table.astype(jnp.float32))
  return out.reshape(h, b, D).transpose(1, 0, 2)


# speedup vs baseline: 1.0146x; 1.0019x over previous
import functools

import jax
import jax.numpy as jnp
from jax import lax
from jax.experimental import pallas as pl
from jax.experimental.pallas import tpu as pltpu
from jax.experimental.pallas import tpu_sc as plsc

NC = 2
NS = 16
NW = NC * NS
CHUNK = 128
D = 128
NB = 5
LOOKAHEAD = 4


@functools.cache
def _emb_kernel(n_idx: int):
  b_per_w = n_idx // NW
  n_chunks = b_per_w // CHUNK
  assert n_chunks % NB == 0
  mesh = plsc.VectorSubcoreMesh(
      core_axis_name="c", subcore_axis_name="s", num_cores=NC, num_subcores=NS
  )

  @functools.partial(
      pl.kernel,
      out_type=jax.ShapeDtypeStruct((n_idx, D), jnp.float32),
      mesh=mesh,
      scratch_types=[
          pltpu.VMEM((b_per_w,), jnp.int32),
          pltpu.VMEM((NB, CHUNK, D), jnp.float32),
          [pltpu.SemaphoreType.DMA for _ in range(NB)],
          [pltpu.SemaphoreType.DMA for _ in range(NB)],
      ],
  )
  def k(words_hbm, table_hbm, out_hbm, idx_v, rows_v, gsems, osems):
    wid = lax.axis_index("s") * NC + lax.axis_index("c")
    base = wid * b_per_w
    pltpu.sync_copy(words_hbm.at[pl.ds(base, b_per_w)], idx_v)

    def fire_gather(c, slot):
      pltpu.async_copy(
          table_hbm.at[idx_v.at[pl.ds(c * CHUNK, CHUNK)]],
          rows_v.at[slot],
          gsems[slot],
      )

    def wait_write(slot):
      pltpu.make_async_copy(
          table_hbm.at[pl.ds(0, CHUNK)], rows_v.at[slot], osems[slot]
      ).wait()

    for c in range(LOOKAHEAD):
      fire_gather(c, c % NB)

    def outer(o, _):
      for b in range(NB):
        j = o * NB + b
        pltpu.make_async_copy(
            table_hbm.at[pl.ds(0, CHUNK)], rows_v.at[b], gsems[b]
        ).wait()
        pltpu.async_copy(
            rows_v.at[b],
            out_hbm.at[pl.ds(base + j * CHUNK, CHUNK)],
            osems[b],
        )
        nb = (b + LOOKAHEAD) % NB

        @pl.when(j + LOOKAHEAD < n_chunks)
        def _():
          @pl.when(j + LOOKAHEAD >= NB)
          def _():
            wait_write(nb)

          fire_gather(j + LOOKAHEAD, nb)

      return 0

    lax.fori_loop(0, n_chunks // NB, outer, 0)

    for b in range(NB):
      wait_write(b)

  return k


def kernel(words, table):
  b, h = words.shape
  idx = words.T.reshape(-1).astype(jnp.int32)
  out = _emb_kernel(idx.shape[0])(idx, table.astype(jnp.float32))
  return out.reshape(h, b, D).transpose(1, 0, 2)
